# baseline (speedup 1.0000x reference)
import functools

import jax
import jax.numpy as jnp
from jax.experimental import pallas as pl
from jax.experimental.pallas import tpu as pltpu

_NSTREAM = 8


def _dma_body(in_hbm, out_hbm, *sems):
    rows = out_hbm.shape[0]
    chunk = rows // _NSTREAM
    copies = []
    for j in range(_NSTREAM):
        copies.append(pltpu.make_async_copy(
            in_hbm.at[pl.ds(j * chunk, chunk)],
            out_hbm.at[pl.ds(j * chunk, chunk)],
            sems[j]))
    for c in copies:
        c.start()
    for c in copies:
        c.wait()


def kernel(x, emb_table):
    seq_len = x.shape[1]
    dim = emb_table.shape[1]
    return pl.pallas_call(
        _dma_body,
        in_specs=[pl.BlockSpec(memory_space=pltpu.MemorySpace.HBM)],
        out_specs=pl.BlockSpec(memory_space=pltpu.MemorySpace.HBM),
        out_shape=jax.ShapeDtypeStruct((seq_len, dim), emb_table.dtype),
        scratch_shapes=[pltpu.SemaphoreType.DMA] * _NSTREAM,
    )(emb_table)


# speedup vs baseline: 42.7029x; 42.7029x over previous
import jax
import jax.numpy as jnp
from jax.experimental import pallas as pl
from jax.experimental.pallas import tpu as pltpu

_CHUNK_ROWS = 512
_NBUF = 6


def _dma_ring_body(in_hbm, out_hbm, *refs):
    bufs = refs[:_NBUF]
    in_sems = refs[_NBUF:2 * _NBUF]
    out_sems = refs[2 * _NBUF:]
    rows = out_hbm.shape[0]
    n_chunks = rows // _CHUNK_ROWS

    def start_in(j):
        b = j % _NBUF
        return pltpu.make_async_copy(
            in_hbm.at[pl.ds(j * _CHUNK_ROWS, _CHUNK_ROWS)], bufs[b], in_sems[b])

    def start_out(j):
        b = j % _NBUF
        return pltpu.make_async_copy(
            bufs[b], out_hbm.at[pl.ds(j * _CHUNK_ROWS, _CHUNK_ROWS)], out_sems[b])

    in_d = [None] * n_chunks
    out_d = [None] * n_chunks
    in_d[0] = start_in(0)
    in_d[0].start()
    for j in range(n_chunks):
        if j + 1 < n_chunks:
            if j + 1 >= _NBUF:
                out_d[j + 1 - _NBUF].wait()
            in_d[j + 1] = start_in(j + 1)
            in_d[j + 1].start()
        in_d[j].wait()
        out_d[j] = start_out(j)
        out_d[j].start()
    for j in range(max(0, n_chunks - _NBUF), n_chunks):
        out_d[j].wait()


def kernel(x, emb_table):
    seq_len = x.shape[1]
    dim = emb_table.shape[1]
    scratch = [pltpu.VMEM((_CHUNK_ROWS, dim), emb_table.dtype)
               for _ in range(_NBUF)]
    scratch += [pltpu.SemaphoreType.DMA] * (2 * _NBUF)
    return pl.pallas_call(
        _dma_ring_body,
        in_specs=[pl.BlockSpec(memory_space=pltpu.MemorySpace.HBM)],
        out_specs=pl.BlockSpec(memory_space=pltpu.MemorySpace.HBM),
        out_shape=jax.ShapeDtypeStruct((seq_len, dim), emb_table.dtype),
        scratch_shapes=scratch,
    )(emb_table)
